# baseline (speedup 1.0000x reference)
import jax
import jax.numpy as jnp
from jax import lax
from jax.experimental import pallas as pl
from jax.experimental.pallas import tpu as pltpu
from jax.experimental.pallas import tpu_sc as plsc

N = 10000
E = 320000
G = 64
EMB = 256
HALF = 128
NL = 3
HID = 128
ADD = 64

NTILE = 16
CHUNK = 125
NCHUNK = E // NTILE // CHUNK
IDXB = 4
NBLK = NCHUNK // IDXB
RPT = 624
ZROWS = 48
NZ = RPT // ZROWS


def _segsum_body(h0, h1, src4, dst4, zfull, a0, a1,
                 acc, sblk0, sblk1, dblk0, dblk1,
                 rows0, rows1,
                 isem0, isem1, gsem0, gsem1, ssem0, ssem1):
    c = lax.axis_index("c")
    s = lax.axis_index("s")
    row0 = s * RPT
    cb = s * NBLK
    is_last = s == NTILE - 1
    sblk = (sblk0, sblk1)
    dblk = (dblk0, dblk1)
    rows = (rows0, rows1)
    isem = (isem0, isem1)
    gsem = (gsem0, gsem1)
    ssem = (ssem0, ssem1)

    def idx_start(t, p):
        pltpu.make_async_copy(src4.at[cb + t], sblk[p], isem[p]).start()
        pltpu.make_async_copy(dst4.at[cb + t], dblk[p], isem[p]).start()

    def idx_wait(p):
        pltpu.make_async_copy(src4.at[cb], sblk[p], isem[p]).wait()
        pltpu.make_async_copy(dst4.at[cb], dblk[p], isem[p]).wait()

    def run(h_ref, out_ref):
        def g_start(p, k, rb):
            pltpu.make_async_copy(h_ref.at[sblk[p].at[k]], rows[rb],
                                  gsem[rb]).start()

        def g_wait(rb):
            pltpu.make_async_copy(h_ref.at[sblk[0].at[0]], rows[rb],
                                  gsem[rb]).wait()

        def s_start(p, k, rb):
            pltpu.async_copy(rows[rb], acc.at[dblk[p].at[k]], ssem[rb],
                             add=True)

        def s_wait(rb):
            pltpu.make_async_copy(rows[rb], acc.at[dblk[0].at[0]],
                                  ssem[rb]).wait()

        idx_start(0, 0)
        idx_start(1, 1)

        pltpu.sync_copy(zfull.at[pl.ds(row0, RPT)], acc.at[pl.ds(row0, RPT)])

        @pl.when(is_last)
        def _():
            pltpu.sync_copy(zfull.at[pl.ds(N - 16, 16)],
                            acc.at[pl.ds(N - 16, 16)])

        plsc.subcore_barrier()

        idx_wait(0)
        g_start(0, 0, 0)

        def pair_body(i, carry):
            for b in range(2):
                t = 2 * i + b
                p = b
                for k in range(IDXB):
                    rb = k % 2
                    if b == 0 and k == 0:
                        @pl.when(i > 0)
                        def _():
                            s_wait(1 - rb)
                    else:
                        s_wait(1 - rb)
                    if k < IDXB - 1:
                        g_start(p, k + 1, (k + 1) % 2)
                    else:
                        @pl.when(t + 1 < NBLK)
                        def _():
                            idx_wait(1 - p)
                            g_start(1 - p, 0, (k + 1) % 2)
                    g_wait(rb)
                    s_start(p, k, rb)
                    if k == IDXB - 1:
                        @pl.when(t + 2 < NBLK)
                        def _():
                            idx_start(t + 2, p)
            return carry
        lax.fori_loop(0, NBLK // 2, pair_body, 0)
        s_wait(1)

        plsc.subcore_barrier()
        pltpu.sync_copy(acc.at[pl.ds(row0, RPT)], out_ref.at[pl.ds(row0, RPT)])

        @pl.when(is_last)
        def _():
            pltpu.sync_copy(acc.at[pl.ds(N - 16, 16)],
                            out_ref.at[pl.ds(N - 16, 16)])

    @pl.when(c == 0)
    def _():
        run(h0, a0)

    @pl.when(c == 1)
    def _():
        run(h1, a1)


def _build_segsum():
  return pl.kernel(
    _segsum_body,
    out_type=(jax.ShapeDtypeStruct((N, HALF), jnp.float32),
              jax.ShapeDtypeStruct((N, HALF), jnp.float32)),
    mesh=plsc.VectorSubcoreMesh(core_axis_name="c", subcore_axis_name="s",
                                num_cores=2, num_subcores=NTILE),
    scratch_types=[
        pltpu.VMEM_SHARED((N, HALF), jnp.float32),
        pltpu.VMEM((IDXB, CHUNK), jnp.int32),
        pltpu.VMEM((IDXB, CHUNK), jnp.int32),
        pltpu.VMEM((IDXB, CHUNK), jnp.int32),
        pltpu.VMEM((IDXB, CHUNK), jnp.int32),
        pltpu.VMEM((CHUNK, HALF), jnp.float32),
        pltpu.VMEM((CHUNK, HALF), jnp.float32),
        pltpu.SemaphoreType.DMA,
        pltpu.SemaphoreType.DMA,
        pltpu.SemaphoreType.DMA,
        pltpu.SemaphoreType.DMA,
        pltpu.SemaphoreType.DMA,
        pltpu.SemaphoreType.DMA,
    ],
  )


def _enc_body(x_ref, w_ref, b_ref, h0_ref, h1_ref):
    xv = x_ref[...]
    w = w_ref[...]
    dn = (((1,), (1,)), ((), ()))
    h0_ref[...] = lax.dot_general(xv, w[:HALF], dn,
                                  preferred_element_type=jnp.float32) + b_ref[0:1]
    h1_ref[...] = lax.dot_general(xv, w[HALF:], dn,
                                  preferred_element_type=jnp.float32) + b_ref[1:2]


_enc = pl.pallas_call(
    _enc_body,
    out_shape=(jax.ShapeDtypeStruct((N, HALF), jnp.float32),
               jax.ShapeDtypeStruct((N, HALF), jnp.float32)),
)


def _layer_body(h0_ref, h1_ref, a0_ref, a1_ref, w1_ref, b1_ref, w2_ref, b2_ref,
                gb_ref, batch_ref, h0o_ref, h1o_ref, pooled_ref):
    dn = (((1,), (1,)), ((), ()))
    z0 = h0_ref[...] + a0_ref[...]
    z1 = h1_ref[...] + a1_ref[...]
    w1 = w1_ref[...]
    y = lax.dot_general(z0, w1[:, :HALF], dn, preferred_element_type=jnp.float32)
    y = y + lax.dot_general(z1, w1[:, HALF:], dn, preferred_element_type=jnp.float32)
    y = jax.nn.relu(y + b1_ref[...])
    z = lax.dot_general(y, w2_ref[...], dn,
                        preferred_element_type=jnp.float32) + b2_ref[...]
    mu = jnp.mean(z, axis=0, keepdims=True)
    zc = z - mu
    var = jnp.mean(zc * zc, axis=0, keepdims=True)
    hn = jax.nn.relu(zc * (gb_ref[0:1] * lax.rsqrt(var + 1e-5)) + gb_ref[1:2])
    h0o_ref[...] = hn[:, :HALF]
    h1o_ref[...] = hn[:, HALF:]
    bt = batch_ref[...]
    gids = lax.broadcasted_iota(jnp.int32, (G, N), 0)
    S = (gids == bt).astype(jnp.float32)
    cnt = jnp.sum(S, axis=1, keepdims=True)
    ps = lax.dot_general(S, hn, (((1,), (0,)), ((), ())),
                         preferred_element_type=jnp.float32)
    pooled_ref[...] = ps / jnp.maximum(cnt, 1.0)


_layer = pl.pallas_call(
    _layer_body,
    out_shape=(jax.ShapeDtypeStruct((N, HALF), jnp.float32),
               jax.ShapeDtypeStruct((N, HALF), jnp.float32),
               jax.ShapeDtypeStruct((G, EMB), jnp.float32)),
)


def _softmax(logits):
    m = jnp.max(logits, axis=-1, keepdims=True)
    e = jnp.exp(logits - m)
    return e / jnp.sum(e, axis=-1, keepdims=True)


def _tail_body(p0_ref, p1_ref, p2_ref, af_ref, wf1_ref, bf1_ref, wga_ref,
               cga_ref, waa_ref, caa_ref, wf2_ref, bf2_ref, out_ref):
    dn = (((1,), (1,)), ((), ()))
    wf1 = wf1_ref[...]
    t = lax.dot_general(p0_ref[...], wf1[:, :EMB], dn,
                        preferred_element_type=jnp.float32)
    t = t + lax.dot_general(p1_ref[...], wf1[:, EMB:2 * EMB], dn,
                            preferred_element_type=jnp.float32)
    t = t + lax.dot_general(p2_ref[...], wf1[:, 2 * EMB:], dn,
                            preferred_element_type=jnp.float32)
    xg = jax.nn.relu(t + bf1_ref[...])
    sg = jnp.tanh(lax.dot_general(xg, wga_ref[...], dn,
                                  preferred_element_type=jnp.float32))
    xg = xg * _softmax(sg * cga_ref[...])
    af = af_ref[...]
    sa = jnp.tanh(lax.dot_general(af, waa_ref[...], dn,
                                  preferred_element_type=jnp.float32))
    af = af * _softmax(sa * caa_ref[...])
    wf2 = wf2_ref[...]
    out = lax.dot_general(xg, wf2[:, :HID], dn,
                          preferred_element_type=jnp.float32)
    out = out + lax.dot_general(af, wf2[:, HID:], dn,
                                preferred_element_type=jnp.float32)
    out_ref[...] = out + bf2_ref[...]


_tail = pl.pallas_call(
    _tail_body,
    out_shape=jax.ShapeDtypeStruct((G, 1), jnp.float32),
)


def kernel(x, edge_index, batch, additional_feature, W_enc, b_enc, W1, b1, W2,
           b2, gamma, beta, W_fc1, b_fc1, W_ga, ctx_ga, W_aa, ctx_aa, W_fc2,
           b_fc2):
    src4 = edge_index[0].reshape(E // (IDXB * CHUNK), IDXB, CHUNK)
    dst4 = edge_index[1].reshape(E // (IDXB * CHUNK), IDXB, CHUNK)
    zfull = jnp.zeros((N, HALF), jnp.float32)
    b_enc2 = jnp.stack([b_enc[:HALF], b_enc[HALF:]])
    batch2 = batch.reshape(1, N)

    segsum = _build_segsum()
    h0, h1 = _enc(x, W_enc, b_enc2)
    pooled = []
    for l in range(NL):
        a0, a1 = segsum(h0, h1, src4, dst4, zfull)
        h0, h1, p = _layer(h0, h1, a0, a1, W1[l], b1[l].reshape(1, EMB),
                           W2[l], b2[l].reshape(1, EMB),
                           jnp.stack([gamma[l], beta[l]]), batch2)
        pooled.append(p)
    out = _tail(pooled[0], pooled[1], pooled[2], additional_feature,
                W_fc1, b_fc1.reshape(1, HID), W_ga, ctx_ga.reshape(1, HID),
                W_aa, ctx_aa.reshape(1, ADD), W_fc2, b_fc2.reshape(1, 1))
    return out


# speedup vs baseline: 1.1753x; 1.1753x over previous
import functools

import jax
import jax.numpy as jnp
from jax import lax
from jax.experimental import pallas as pl
from jax.experimental.pallas import tpu as pltpu
from jax.experimental.pallas import tpu_sc as plsc

N = 10000
E = 320000
G = 64
EMB = 256
HALF = 128
NL = 3
HID = 128
ADD = 64

NTILE = 16
CHUNK = 125
NCHUNK = E // NTILE // CHUNK
IDXB = 4
NBLK = NCHUNK // IDXB
RPT = 624
ZROWS = 48
NZ = RPT // ZROWS


def _segsum_body(h0, h1, e4, zrows, a0, a1,
                 acc, sblk0, sblk1, dblk0, dblk1,
                 rows0, rows1, zbuf,
                 isem0, isem1, gsem0, gsem1, ssem0, ssem1):
    src4 = e4.at[0]
    dst4 = e4.at[1]
    c = lax.axis_index("c")
    s = lax.axis_index("s")
    row0 = s * RPT
    cb = s * NBLK
    is_last = s == NTILE - 1
    sblk = (sblk0, sblk1)
    dblk = (dblk0, dblk1)
    rows = (rows0, rows1)
    isem = (isem0, isem1)
    gsem = (gsem0, gsem1)
    ssem = (ssem0, ssem1)

    def idx_start(t, p):
        pltpu.make_async_copy(src4.at[cb + t], sblk[p], isem[p]).start()
        pltpu.make_async_copy(dst4.at[cb + t], dblk[p], isem[p]).start()

    def idx_wait(p):
        pltpu.make_async_copy(src4.at[cb], sblk[p], isem[p]).wait()
        pltpu.make_async_copy(dst4.at[cb], dblk[p], isem[p]).wait()

    def run(h_ref, out_ref):
        def g_start(p, k, rb):
            pltpu.make_async_copy(h_ref.at[sblk[p].at[k]], rows[rb],
                                  gsem[rb]).start()

        def g_wait(rb):
            pltpu.make_async_copy(h_ref.at[sblk[0].at[0]], rows[rb],
                                  gsem[rb]).wait()

        def s_start(p, k, rb):
            pltpu.async_copy(rows[rb], acc.at[dblk[p].at[k]], ssem[rb],
                             add=True)

        def s_wait(rb):
            pltpu.make_async_copy(rows[rb], acc.at[dblk[0].at[0]],
                                  ssem[rb]).wait()

        idx_start(0, 0)
        idx_start(1, 1)

        pltpu.sync_copy(zrows, zbuf)
        for t in range(NZ):
            pltpu.sync_copy(zbuf, acc.at[pl.ds(row0 + t * ZROWS, ZROWS)])

        @pl.when(is_last)
        def _():
            pltpu.sync_copy(zbuf.at[pl.ds(0, 16)], acc.at[pl.ds(N - 16, 16)])

        plsc.subcore_barrier()

        idx_wait(0)
        g_start(0, 0, 0)

        def pair_body(i, carry):
            for b in range(2):
                t = 2 * i + b
                p = b
                for k in range(IDXB):
                    rb = k % 2
                    if b == 0 and k == 0:
                        @pl.when(i > 0)
                        def _():
                            s_wait(1 - rb)
                    else:
                        s_wait(1 - rb)
                    if k < IDXB - 1:
                        g_start(p, k + 1, (k + 1) % 2)
                    else:
                        @pl.when(t + 1 < NBLK)
                        def _():
                            idx_wait(1 - p)
                            g_start(1 - p, 0, (k + 1) % 2)
                    g_wait(rb)
                    s_start(p, k, rb)
                    if k == IDXB - 1:
                        @pl.when(t + 2 < NBLK)
                        def _():
                            idx_start(t + 2, p)
            return carry
        lax.fori_loop(0, NBLK // 2, pair_body, 0)
        s_wait(1)

        plsc.subcore_barrier()
        for t in range(NZ):
            pltpu.sync_copy(acc.at[pl.ds(row0 + t * ZROWS, ZROWS)], zbuf)
            pltpu.sync_copy(zbuf, out_ref.at[pl.ds(row0 + t * ZROWS, ZROWS)])

        @pl.when(is_last)
        def _():
            pltpu.sync_copy(acc.at[pl.ds(N - 16, 16)], zbuf.at[pl.ds(0, 16)])
            pltpu.sync_copy(zbuf.at[pl.ds(0, 16)],
                            out_ref.at[pl.ds(N - 16, 16)])

    @pl.when(c == 0)
    def _():
        run(h0, a0)

    @pl.when(c == 1)
    def _():
        run(h1, a1)


def _build_segsum():
  return pl.kernel(
    _segsum_body,
    out_type=(jax.ShapeDtypeStruct((N, HALF), jnp.float32),
              jax.ShapeDtypeStruct((N, HALF), jnp.float32)),
    mesh=plsc.VectorSubcoreMesh(core_axis_name="c", subcore_axis_name="s",
                                num_cores=2, num_subcores=NTILE),
    scratch_types=[
        pltpu.VMEM_SHARED((N, HALF), jnp.float32),
        pltpu.VMEM((IDXB, CHUNK), jnp.int32),
        pltpu.VMEM((IDXB, CHUNK), jnp.int32),
        pltpu.VMEM((IDXB, CHUNK), jnp.int32),
        pltpu.VMEM((IDXB, CHUNK), jnp.int32),
        pltpu.VMEM((CHUNK, HALF), jnp.float32),
        pltpu.VMEM((CHUNK, HALF), jnp.float32),
        pltpu.VMEM((ZROWS, HALF), jnp.float32),
        pltpu.SemaphoreType.DMA,
        pltpu.SemaphoreType.DMA,
        pltpu.SemaphoreType.DMA,
        pltpu.SemaphoreType.DMA,
        pltpu.SemaphoreType.DMA,
        pltpu.SemaphoreType.DMA,
    ],
  )


def _enc_body(x_ref, w_ref, b_ref, h0_ref, h1_ref):
    xv = x_ref[...]
    w = w_ref[...]
    dn = (((1,), (1,)), ((), ()))
    h0_ref[...] = lax.dot_general(xv, w[:HALF], dn,
                                  preferred_element_type=jnp.float32) + b_ref[:, :HALF]
    h1_ref[...] = lax.dot_general(xv, w[HALF:], dn,
                                  preferred_element_type=jnp.float32) + b_ref[:, HALF:]


_enc = pl.pallas_call(
    _enc_body,
    out_shape=(jax.ShapeDtypeStruct((N, HALF), jnp.float32),
               jax.ShapeDtypeStruct((N, HALF), jnp.float32)),
)


def _layer_body(l, h0_ref, h1_ref, a0_ref, a1_ref, w1_ref, b1_ref, w2_ref,
                b2_ref, gamma_ref, beta_ref, batch_ref,
                h0o_ref, h1o_ref, pooled_ref):
    dn = (((1,), (1,)), ((), ()))
    z0 = h0_ref[...] + a0_ref[...]
    z1 = h1_ref[...] + a1_ref[...]
    w1 = w1_ref[l]
    y = lax.dot_general(z0, w1[:, :HALF], dn, preferred_element_type=jnp.float32)
    y = y + lax.dot_general(z1, w1[:, HALF:], dn, preferred_element_type=jnp.float32)
    y = jax.nn.relu(y + b1_ref[l:l + 1])
    z = lax.dot_general(y, w2_ref[l], dn,
                        preferred_element_type=jnp.float32) + b2_ref[l:l + 1]
    mu = jnp.mean(z, axis=0, keepdims=True)
    zc = z - mu
    var = jnp.mean(zc * zc, axis=0, keepdims=True)
    hn = jax.nn.relu(zc * (gamma_ref[l:l + 1] * lax.rsqrt(var + 1e-5))
                     + beta_ref[l:l + 1])
    h0o_ref[...] = hn[:, :HALF]
    h1o_ref[...] = hn[:, HALF:]
    bt = batch_ref[...]
    gids = lax.broadcasted_iota(jnp.int32, (G, N), 0)
    S = (gids == bt).astype(jnp.float32)
    cnt = jnp.sum(S, axis=1, keepdims=True)
    ps = lax.dot_general(S, hn, (((1,), (0,)), ((), ())),
                         preferred_element_type=jnp.float32)
    pooled_ref[...] = ps / jnp.maximum(cnt, 1.0)


_layers = [
    pl.pallas_call(
        functools.partial(_layer_body, l),
        out_shape=(jax.ShapeDtypeStruct((N, HALF), jnp.float32),
                   jax.ShapeDtypeStruct((N, HALF), jnp.float32),
                   jax.ShapeDtypeStruct((G, EMB), jnp.float32)),
    )
    for l in range(NL)
]


def _softmax(logits):
    m = jnp.max(logits, axis=-1, keepdims=True)
    e = jnp.exp(logits - m)
    return e / jnp.sum(e, axis=-1, keepdims=True)


def _tail_body(p0_ref, p1_ref, p2_ref, af_ref, wf1_ref, bf1_ref, wga_ref,
               cga_ref, waa_ref, caa_ref, wf2_ref, bf2_ref, out_ref):
    dn = (((1,), (1,)), ((), ()))
    wf1 = wf1_ref[...]
    t = lax.dot_general(p0_ref[...], wf1[:, :EMB], dn,
                        preferred_element_type=jnp.float32)
    t = t + lax.dot_general(p1_ref[...], wf1[:, EMB:2 * EMB], dn,
                            preferred_element_type=jnp.float32)
    t = t + lax.dot_general(p2_ref[...], wf1[:, 2 * EMB:], dn,
                            preferred_element_type=jnp.float32)
    xg = jax.nn.relu(t + bf1_ref[...])
    sg = jnp.tanh(lax.dot_general(xg, wga_ref[...], dn,
                                  preferred_element_type=jnp.float32))
    xg = xg * _softmax(sg * cga_ref[...])
    af = af_ref[...]
    sa = jnp.tanh(lax.dot_general(af, waa_ref[...], dn,
                                  preferred_element_type=jnp.float32))
    af = af * _softmax(sa * caa_ref[...])
    wf2 = wf2_ref[...]
    out = lax.dot_general(xg, wf2[:, :HID], dn,
                          preferred_element_type=jnp.float32)
    out = out + lax.dot_general(af, wf2[:, HID:], dn,
                                preferred_element_type=jnp.float32)
    out_ref[...] = out + bf2_ref[...]


_tail = pl.pallas_call(
    _tail_body,
    out_shape=jax.ShapeDtypeStruct((G, 1), jnp.float32),
)


def kernel(x, edge_index, batch, additional_feature, W_enc, b_enc, W1, b1, W2,
           b2, gamma, beta, W_fc1, b_fc1, W_ga, ctx_ga, W_aa, ctx_aa, W_fc2,
           b_fc2):
    e4 = edge_index.reshape(2, E // (IDXB * CHUNK), IDXB, CHUNK)
    zrows = jnp.zeros((ZROWS, HALF), jnp.float32)
    batch2 = batch.reshape(1, N)

    segsum = _build_segsum()
    h0, h1 = _enc(x, W_enc, b_enc.reshape(1, EMB))
    pooled = []
    for l in range(NL):
        a0, a1 = segsum(h0, h1, e4, zrows)
        h0, h1, p = _layers[l](h0, h1, a0, a1, W1, b1, W2, b2, gamma, beta,
                               batch2)
        pooled.append(p)
    out = _tail(pooled[0], pooled[1], pooled[2], additional_feature,
                W_fc1, b_fc1.reshape(1, HID), W_ga, ctx_ga.reshape(1, HID),
                W_aa, ctx_aa.reshape(1, ADD), W_fc2, b_fc2.reshape(1, 1))
    return out
